# baseline (speedup 1.0000x reference)
import functools
import jax
import jax.numpy as jnp
from jax import lax
from jax.experimental import pallas as pl
from jax.experimental.pallas import tpu as pltpu
from jax.experimental.pallas import tpu_sc as plsc

NC, NS, L = 2, 16, 16
NW = NC * NS
D = 64
BATCH, SEQ = 4096, 200
BPW = BATCH // NW
NBUF = 2

_mesh = plsc.VectorSubcoreMesh(core_axis_name="c", subcore_axis_name="s")


@functools.partial(
    pl.kernel,
    out_type=jax.ShapeDtypeStruct((SEQ, D, BATCH), jnp.float32),
    mesh=_mesh,
    scratch_types=[
        pltpu.VMEM((SEQ, BPW), jnp.int32),
        pltpu.VMEM((SEQ, D), jnp.float32),
        pltpu.VMEM((NBUF, BPW, D), jnp.float32),
        pltpu.VMEM((NBUF, D, BPW), jnp.float32),
        pltpu.SemaphoreType.DMA,
        pltpu.SemaphoreType.DMA,
    ],
    compiler_params=pltpu.CompilerParams(use_tc_tiling_on_sc=False,
                                         needs_layout_passes=False),
)
def _emb_kernel(tok_hbm, table_hbm, pe_hbm, out_hbm,
                tok_v, pe_v, rows_v, out_v, gsem, ssem):
    wid = lax.axis_index("s") * NC + lax.axis_index("c")
    b0 = wid * BPW
    pltpu.sync_copy(pe_hbm.at[pl.ds(0, SEQ)], pe_v)
    pltpu.sync_copy(tok_hbm.at[pl.ds(0, SEQ), pl.ds(b0, BPW)], tok_v)

    def fire_gather(s, bi):
        pltpu.async_copy(table_hbm.at[tok_v.at[s]], rows_v.at[bi], gsem)

    def fire_store(s, bi):
        pltpu.async_copy(out_v.at[bi],
                         out_hbm.at[s, pl.ds(0, D), pl.ds(b0, BPW)], ssem)

    for bi in range(NBUF):
        fire_gather(bi, bi)

    lanes = lax.iota(jnp.int32, L)
    ridx = [lanes + k * L for k in range(D // L)]

    def outer(t, _):
        for bi in range(NBUF):
            s = t * NBUF + bi
            pltpu.make_async_copy(
                table_hbm.at[pl.ds(0, BPW)], rows_v.at[bi], gsem).wait()

            @pl.when(s >= NBUF)
            def _():
                pltpu.make_async_copy(
                    out_v.at[bi],
                    out_hbm.at[0, pl.ds(0, D), pl.ds(b0, BPW)], ssem).wait()

            rb, ob = rows_v.at[bi], out_v.at[bi]
            pv = [pe_v[s, pl.ds(k * L, L)] for k in range(D // L)]

            def row(bb, _):
                cidx = jnp.full((L,), bb, jnp.int32)
                for k in range(D // L):
                    x = rb[bb, pl.ds(k * L, L)] * 8.0 + pv[k]
                    plsc.store_scatter(ob, [ridx[k], cidx], x)
                return 0
            lax.fori_loop(0, BPW, row, 0)

            fire_store(s, bi)

            @pl.when(s + NBUF < SEQ)
            def _():
                fire_gather(s + NBUF, bi)
        return 0

    lax.fori_loop(0, SEQ // NBUF, outer, 0)

    for bi in range(NBUF):
        pltpu.make_async_copy(
            out_v.at[bi],
            out_hbm.at[0, pl.ds(0, D), pl.ds(b0, BPW)], ssem).wait()


def kernel(token_sequences, embedding_weight, positional_embedding):
    tok_t = token_sequences.T
    pe = positional_embedding.reshape(positional_embedding.shape[1], D)
    out = _emb_kernel(tok_t, embedding_weight, pe)
    return out.transpose(2, 0, 1)


# speedup vs baseline: 1.1569x; 1.1569x over previous
import functools
import jax
import jax.numpy as jnp
from jax import lax
from jax.experimental import pallas as pl
from jax.experimental.pallas import tpu as pltpu
from jax.experimental.pallas import tpu_sc as plsc

NC, NS, L = 2, 16, 16
NW = NC * NS
D = 64
BATCH, SEQ = 4096, 200
BPW = BATCH // NW
NBUF = 2
TD = D // 8

_mesh = plsc.VectorSubcoreMesh(core_axis_name="c", subcore_axis_name="s")


@functools.partial(
    pl.kernel,
    out_type=jax.ShapeDtypeStruct((SEQ, TD, NW, 8, BPW), jnp.float32),
    mesh=_mesh,
    scratch_types=[
        pltpu.VMEM((SEQ, BPW), jnp.int32),
        pltpu.VMEM((SEQ, D), jnp.float32),
        pltpu.VMEM((NBUF, BPW, D), jnp.float32),
        pltpu.VMEM((NBUF, TD, 8, BPW), jnp.float32),
        pltpu.SemaphoreType.DMA,
        pltpu.SemaphoreType.DMA,
    ],
    compiler_params=pltpu.CompilerParams(use_tc_tiling_on_sc=False,
                                         needs_layout_passes=False),
)
def _emb_kernel(tok_hbm, table_hbm, pe_hbm, out_hbm,
                tok_v, pe_v, rows_v, out_v, gsem, ssem):
    wid = lax.axis_index("s") * NC + lax.axis_index("c")
    b0 = wid * BPW
    pltpu.sync_copy(pe_hbm.at[pl.ds(0, SEQ)], pe_v)
    pltpu.sync_copy(tok_hbm.at[pl.ds(0, SEQ), pl.ds(b0, BPW)], tok_v)

    def fire_gather(s, bi):
        pltpu.async_copy(table_hbm.at[tok_v.at[s]], rows_v.at[bi], gsem)

    def fire_store(s, bi):
        pltpu.async_copy(
            out_v.at[bi],
            out_hbm.at[s, pl.ds(0, TD), wid, pl.ds(0, 8), pl.ds(0, BPW)],
            ssem)

    for bi in range(NBUF):
        fire_gather(bi, bi)

    lanes = lax.iota(jnp.int32, L)
    cbs = [(lanes >> 3) + 2 * k for k in range(D // L)]
    c8s = lanes & 7

    def outer(t, _):
        for bi in range(NBUF):
            s = t * NBUF + bi
            pltpu.make_async_copy(
                table_hbm.at[pl.ds(0, BPW)], rows_v.at[bi], gsem).wait()

            @pl.when(s >= NBUF)
            def _():
                pltpu.make_async_copy(
                    out_v.at[bi],
                    out_hbm.at[0, pl.ds(0, TD), 0, pl.ds(0, 8),
                               pl.ds(0, BPW)],
                    ssem).wait()

            rb, ob = rows_v.at[bi], out_v.at[bi]
            pv = [pe_v[s, pl.ds(k * L, L)] for k in range(D // L)]

            def row(bb, _):
                cidx = jnp.full((L,), bb, jnp.int32)
                for k in range(D // L):
                    x = rb[bb, pl.ds(k * L, L)] * 8.0 + pv[k]
                    plsc.store_scatter(ob, [cbs[k], c8s, cidx], x)
                return 0
            lax.fori_loop(0, BPW, row, 0)

            fire_store(s, bi)

            @pl.when(s + NBUF < SEQ)
            def _():
                fire_gather(s + NBUF, bi)
        return 0

    lax.fori_loop(0, SEQ // NBUF, outer, 0)

    for bi in range(NBUF):
        pltpu.make_async_copy(
            out_v.at[bi],
            out_hbm.at[0, pl.ds(0, TD), 0, pl.ds(0, 8), pl.ds(0, BPW)],
            ssem).wait()


def kernel(token_sequences, embedding_weight, positional_embedding):
    tok_t = token_sequences.T
    pe = positional_embedding.reshape(positional_embedding.shape[1], D)
    out = _emb_kernel(tok_t, embedding_weight, pe)
    return out.transpose(2, 4, 0, 1, 3).reshape(BATCH, SEQ, D)


# speedup vs baseline: 1.2669x; 1.0951x over previous
import functools
import jax
import jax.numpy as jnp
from jax import lax
from jax.experimental import pallas as pl
from jax.experimental.pallas import tpu as pltpu
from jax.experimental.pallas import tpu_sc as plsc

NC, NS, L = 2, 16, 16
NW = NC * NS
D = 64
BATCH, SEQ = 4096, 200
BPW = BATCH // NW
NBUF = 2
TD = D // 8

_mesh = plsc.VectorSubcoreMesh(core_axis_name="c", subcore_axis_name="s")


@functools.partial(
    pl.kernel,
    out_type=jax.ShapeDtypeStruct((SEQ, TD, NW, 8, BPW), jnp.float32),
    mesh=_mesh,
    scratch_types=[
        pltpu.VMEM((SEQ, BPW), jnp.int32),
        pltpu.VMEM((SEQ, D), jnp.float32),
        pltpu.VMEM((NBUF, BPW, D), jnp.float32),
        pltpu.VMEM((NBUF, BPW, D + 1), jnp.float32),
        pltpu.VMEM((NBUF, TD, 8, BPW), jnp.float32),
        pltpu.SemaphoreType.DMA,
        pltpu.SemaphoreType.DMA,
    ],
    compiler_params=pltpu.CompilerParams(use_tc_tiling_on_sc=False,
                                         needs_layout_passes=False),
)
def _emb_kernel(tok_hbm, table_hbm, pe_hbm, out_hbm,
                tok_v, pe_v, rows_v, rows2_v, out_v, gsem, ssem):
    wid = lax.axis_index("s") * NC + lax.axis_index("c")
    b0 = wid * BPW
    pltpu.sync_copy(pe_hbm.at[pl.ds(0, SEQ)], pe_v)
    pltpu.sync_copy(tok_hbm.at[pl.ds(0, SEQ), pl.ds(b0, BPW)], tok_v)

    def fire_gather(s, bi):
        pltpu.async_copy(table_hbm.at[tok_v.at[s]], rows_v.at[bi], gsem)

    def fire_store(s, bi):
        pltpu.async_copy(
            out_v.at[bi],
            out_hbm.at[s, pl.ds(0, TD), wid, pl.ds(0, 8), pl.ds(0, BPW)],
            ssem)

    for bi in range(NBUF):
        fire_gather(bi, bi)

    lanes = lax.iota(jnp.int32, L)
    cols = [lanes + k * L for k in range(D // L)]
    grps = [lanes + g * L for g in range(BPW // L)]

    def outer(t, _):
        for bi in range(NBUF):
            s = t * NBUF + bi
            pltpu.make_async_copy(
                table_hbm.at[pl.ds(0, BPW)], rows_v.at[bi], gsem).wait()

            @pl.when(s >= NBUF)
            def _():
                pltpu.make_async_copy(
                    out_v.at[bi],
                    out_hbm.at[0, pl.ds(0, TD), 0, pl.ds(0, 8),
                               pl.ds(0, BPW)],
                    ssem).wait()

            rb, r2, ob = rows_v.at[bi], rows2_v.at[bi], out_v.at[bi]
            pv = [pe_v[s, pl.ds(k * L, L)] for k in range(D // L)]

            def row(bb, _):
                ridx = jnp.full((L,), bb, jnp.int32)
                for k in range(D // L):
                    x = rb[bb, pl.ds(k * L, L)] * 8.0 + pv[k]
                    plsc.store_scatter(r2, [ridx, cols[k]], x)
                return 0
            lax.fori_loop(0, BPW, row, 0)

            def col(c, _):
                cb = c >> 3
                c8 = c & 7
                cidx = jnp.full((L,), c, jnp.int32)
                for g in range(BPW // L):
                    x = plsc.load_gather(r2, [grps[g], cidx])
                    ob[cb, c8, pl.ds(g * L, L)] = x
                return 0
            lax.fori_loop(0, D, col, 0)

            fire_store(s, bi)

            @pl.when(s + NBUF < SEQ)
            def _():
                fire_gather(s + NBUF, bi)
        return 0

    lax.fori_loop(0, SEQ // NBUF, outer, 0)

    for bi in range(NBUF):
        pltpu.make_async_copy(
            out_v.at[bi],
            out_hbm.at[0, pl.ds(0, TD), 0, pl.ds(0, 8), pl.ds(0, BPW)],
            ssem).wait()


def kernel(token_sequences, embedding_weight, positional_embedding):
    tok_t = token_sequences.T
    pe = positional_embedding.reshape(positional_embedding.shape[1], D)
    out = _emb_kernel(tok_t, embedding_weight, pe)
    return out.transpose(2, 4, 0, 1, 3).reshape(BATCH, SEQ, D)


# speedup vs baseline: 1.2671x; 1.0002x over previous
import functools
import jax
import jax.numpy as jnp
from jax import lax
from jax.experimental import pallas as pl
from jax.experimental.pallas import tpu as pltpu
from jax.experimental.pallas import tpu_sc as plsc

NC, NS, L = 2, 16, 16
NW = NC * NS
D = 64
BATCH, SEQ = 4096, 200
BPW = BATCH // NW
NBUF = 2
TD = D // 8

_mesh = plsc.VectorSubcoreMesh(core_axis_name="c", subcore_axis_name="s")


@functools.partial(
    pl.kernel,
    out_type=jax.ShapeDtypeStruct((SEQ, TD, NW, 8, BPW), jnp.float32),
    mesh=_mesh,
    scratch_types=[
        pltpu.VMEM((SEQ, BPW), jnp.int32),
        pltpu.VMEM((SEQ, D), jnp.float32),
        pltpu.VMEM((NBUF, BPW, D), jnp.float32),
        pltpu.VMEM((NBUF, BPW, D + 1), jnp.float32),
        pltpu.VMEM((NBUF, TD, 8, BPW), jnp.float32),
        pltpu.SemaphoreType.DMA,
        pltpu.SemaphoreType.DMA,
    ],
    compiler_params=pltpu.CompilerParams(use_tc_tiling_on_sc=False,
                                         needs_layout_passes=False),
)
def _emb_kernel(tok_hbm, table_hbm, pe_hbm, out_hbm,
                tok_v, pe_v, rows_v, rows2_v, out_v, gsem, ssem):
    wid = lax.axis_index("s") * NC + lax.axis_index("c")
    b0 = wid * BPW
    pltpu.sync_copy(pe_hbm.at[pl.ds(0, SEQ)], pe_v)
    pltpu.sync_copy(tok_hbm.at[pl.ds(0, SEQ), pl.ds(b0, BPW)], tok_v)

    def fire_gather(s, bi):
        pltpu.async_copy(table_hbm.at[tok_v.at[s]], rows_v.at[bi], gsem)

    def fire_store(s, bi):
        pltpu.async_copy(
            out_v.at[bi],
            out_hbm.at[s, pl.ds(0, TD), wid, pl.ds(0, 8), pl.ds(0, BPW)],
            ssem)

    for bi in range(NBUF):
        fire_gather(bi, bi)

    lanes = lax.iota(jnp.int32, L)
    cols = [lanes + k * L for k in range(D // L)]
    grps = [lanes + g * L for g in range(BPW // L)]

    def outer(t, _):
        for bi in range(NBUF):
            s = t * NBUF + bi
            with jax.named_scope("drain_gather"):
                pltpu.make_async_copy(
                    table_hbm.at[pl.ds(0, BPW)], rows_v.at[bi], gsem).wait()

            with jax.named_scope("wait_store"):
                @pl.when(s >= NBUF)
                def _():
                    pltpu.make_async_copy(
                        out_v.at[bi],
                        out_hbm.at[0, pl.ds(0, TD), 0, pl.ds(0, 8),
                                   pl.ds(0, BPW)],
                        ssem).wait()

            rb, r2, ob = rows_v.at[bi], rows2_v.at[bi], out_v.at[bi]
            pv = [pe_v[s, pl.ds(k * L, L)] for k in range(D // L)]

            def row(bb, _):
                ridx = jnp.full((L,), bb, jnp.int32)
                for k in range(D // L):
                    x = rb[bb, pl.ds(k * L, L)] * 8.0 + pv[k]
                    plsc.store_scatter(r2, [ridx, cols[k]], x)
                return 0
            with jax.named_scope("pass1"):
                lax.fori_loop(0, BPW, row, 0)

            def col(c, _):
                cb = c >> 3
                c8 = c & 7
                cidx = jnp.full((L,), c, jnp.int32)
                for g in range(BPW // L):
                    x = plsc.load_gather(r2, [grps[g], cidx])
                    ob[cb, c8, pl.ds(g * L, L)] = x
                return 0
            with jax.named_scope("pass2"):
                lax.fori_loop(0, D, col, 0)

            with jax.named_scope("fire_store"):
                fire_store(s, bi)

            @pl.when(s + NBUF < SEQ)
            def _():
                fire_gather(s + NBUF, bi)
        return 0

    lax.fori_loop(0, SEQ // NBUF, outer, 0)

    for bi in range(NBUF):
        pltpu.make_async_copy(
            out_v.at[bi],
            out_hbm.at[0, pl.ds(0, TD), 0, pl.ds(0, 8), pl.ds(0, BPW)],
            ssem).wait()


def kernel(token_sequences, embedding_weight, positional_embedding):
    tok_t = token_sequences.T
    pe = positional_embedding.reshape(positional_embedding.shape[1], D)
    out = _emb_kernel(tok_t, embedding_weight, pe)
    return out.transpose(2, 4, 0, 1, 3).reshape(BATCH, SEQ, D)
